# baseline (speedup 1.0000x reference)
import functools

import jax
import jax.numpy as jnp
from jax import lax
from jax.experimental import pallas as pl
from jax.experimental.pallas import tpu as pltpu
from jax.experimental.pallas import tpu_sc as plsc

HID = 128
POSW = 16
HPW = HID + POSW
CH = 128
NC = 2
NS = 16
NW = NC * NS


def _silu(x):
    return x * jax.nn.sigmoid(x)


def _sc_gather(hp, rc):
    n = hp.shape[0]
    nb = rc.shape[0]
    e = nb * CH
    base_ct = nb // NW
    rem = nb - base_ct * NW
    mesh = plsc.VectorSubcoreMesh(core_axis_name="c", subcore_axis_name="s",
                                  num_cores=NC, num_subcores=NS)

    def body(hp_hbm, rc_hbm, hpr_hbm, hpc_hbm,
             idx_v, hpr_v, hpc_v, sem):
        cid = lax.axis_index("c")
        sid = lax.axis_index("s")
        wid = sid * NC + cid
        myct = base_ct + (wid < rem).astype(jnp.int32)

        def chunk(k, carry):
            r = k * NW + wid
            base = r * CH
            pltpu.sync_copy(rc_hbm.at[r], idx_v)
            a1 = pltpu.async_copy(hp_hbm.at[idx_v.at[0]], hpr_v, sem)
            a2 = pltpu.async_copy(hp_hbm.at[idx_v.at[1]], hpc_v, sem)
            a1.wait()
            a2.wait()
            pltpu.sync_copy(hpr_v, hpr_hbm.at[pl.ds(base, CH)])
            pltpu.sync_copy(hpc_v, hpc_hbm.at[pl.ds(base, CH)])
            return carry

        lax.fori_loop(0, myct, chunk, 0)

    f = pl.kernel(
        body,
        out_type=[
            jax.ShapeDtypeStruct((e, HPW), jnp.float32),
            jax.ShapeDtypeStruct((e, HPW), jnp.float32),
        ],
        mesh=mesh,
        compiler_params=pltpu.CompilerParams(use_tc_tiling_on_sc=False),
        scratch_types=[
            pltpu.VMEM((2, CH), jnp.int32),
            pltpu.VMEM((CH, HPW), jnp.float32),
            pltpu.VMEM((CH, HPW), jnp.float32),
            pltpu.SemaphoreType.DMA,
        ],
    )
    return f(hp, rc)


def _sc_scatter(pay, row1, zeros_p):
    n = zeros_p.shape[0]
    e = row1.shape[0]
    nb = e // CH
    base_ct = nb // NW
    rem = nb - base_ct * NW
    rpt = (n // NS) // 8 * 8
    ex = n - rpt * NS
    mesh = plsc.VectorSubcoreMesh(core_axis_name="c", subcore_axis_name="s",
                                  num_cores=NC, num_subcores=NS)

    def body(pay_hbm, row_hbm, zp_hbm, aggp_hbm,
             idx_v, pay_v, acc_sh, sem):
        cid = lax.axis_index("c")
        sid = lax.axis_index("s")
        wid = sid * NC + cid
        myct = base_ct + (wid < rem).astype(jnp.int32)
        r0 = sid * rpt

        pltpu.sync_copy(zp_hbm.at[pl.ds(r0, rpt)], acc_sh.at[pl.ds(r0, rpt)])
        if ex:
            @pl.when(sid == 0)
            def _():
                pltpu.sync_copy(zp_hbm.at[pl.ds(rpt * NS, ex)],
                                acc_sh.at[pl.ds(rpt * NS, ex)])
        plsc.subcore_barrier()

        def chunk(k, carry):
            r = k * NW + wid
            base = r * CH
            pltpu.sync_copy(row_hbm.at[pl.ds(base, CH)], idx_v)
            pltpu.sync_copy(pay_hbm.at[pl.ds(base, CH)], pay_v)
            pltpu.sync_copy(pay_v, acc_sh.at[idx_v], add=True)
            return carry

        lax.fori_loop(0, myct, chunk, 0)
        plsc.subcore_barrier()

        pltpu.sync_copy(acc_sh.at[pl.ds(r0, rpt)],
                        aggp_hbm.at[cid, pl.ds(r0, rpt)])
        if ex:
            @pl.when(sid == 0)
            def _():
                pltpu.sync_copy(acc_sh.at[pl.ds(rpt * NS, ex)],
                                aggp_hbm.at[cid, pl.ds(rpt * NS, ex)])

    f = pl.kernel(
        body,
        out_type=jax.ShapeDtypeStruct((NC, n, HPW), jnp.float32),
        mesh=mesh,
        compiler_params=pltpu.CompilerParams(use_tc_tiling_on_sc=False),
        scratch_types=[
            pltpu.VMEM((CH,), jnp.int32),
            pltpu.VMEM((CH, HPW), jnp.float32),
            pltpu.VMEM_SHARED((n, HPW), jnp.float32),
            pltpu.SemaphoreType.DMA,
        ],
    )
    return f(pay, row1, zeros_p)


def _emb_body(x_ref, pos_ref, w0, b0, w1, b1, w2, b2, out_ref):
    h = _silu(x_ref[...] @ w0[...] + b0[...])
    h = _silu(h @ w1[...] + b1[...])
    h = h @ w2[...] + b2[...]
    out_ref[...] = jnp.concatenate([h, pos_ref[...]], axis=1)


def _emb_call(x, pos16, p, bn):
    n, af = x.shape
    grid = (n // bn,)
    full = lambda shape: pl.BlockSpec(shape, lambda i: (0, 0))
    return pl.pallas_call(
        _emb_body,
        grid=grid,
        in_specs=[
            pl.BlockSpec((bn, af), lambda i: (i, 0)),
            pl.BlockSpec((bn, POSW), lambda i: (i, 0)),
            full((af, HID)), full((1, HID)),
            full((HID, HID)), full((1, HID)),
            full((HID, HID)), full((1, HID)),
        ],
        out_specs=pl.BlockSpec((bn, HPW), lambda i: (i, 0)),
        out_shape=jax.ShapeDtypeStruct((n, HPW), jnp.float32),
    )(x, pos16, p['w0'], p['b0'], p['w1'], p['b1'], p['w2'], p['b2'])


def _edge_body(hpr_ref, hpc_ref,
               w1a, w1b, w1d, b1, w2, b2, wc1, bc1, wc2t,
               pay_ref):
    dot = lambda a, w: lax.dot(a, w, preferred_element_type=jnp.float32)
    xr = hpr_ref[...]
    xc = hpc_ref[...]
    hr = xr[:, :HID].astype(jnp.bfloat16)
    hc = xc[:, :HID].astype(jnp.bfloat16)
    rel = xr[:, HID:] - xc[:, HID:]
    d2 = jnp.sum(rel * rel, axis=1, keepdims=True)
    t = dot(hr, w1a[...]) + dot(hc, w1b[...]) + d2 * w1d[...] + b1[...]
    m1 = _silu(t).astype(jnp.bfloat16)
    m = _silu(dot(m1, w2[...]) + b2[...])
    c1 = _silu(dot(m.astype(jnp.bfloat16), wc1[...]) + bc1[...])
    cw = jnp.sum(c1 * wc2t[...], axis=1, keepdims=True)
    be = rel.shape[0]
    cnt1 = (lax.broadcasted_iota(jnp.int32, (be, POSW), 1) == 3).astype(jnp.float32)
    pay_ref[...] = jnp.concatenate([m, cw * rel + cnt1], axis=1)


def _edge_call(hpr, hpc, wp, be):
    e = hpr.shape[0]
    grid = (e // be,)
    full = lambda shape: pl.BlockSpec(shape, lambda i: (0, 0))
    return pl.pallas_call(
        _edge_body,
        grid=grid,
        in_specs=[
            pl.BlockSpec((be, HPW), lambda i: (i, 0)),
            pl.BlockSpec((be, HPW), lambda i: (i, 0)),
            full((HID, HID)), full((HID, HID)), full((1, HID)), full((1, HID)),
            full((HID, HID)), full((1, HID)),
            full((HID, HID)), full((1, HID)), full((1, HID)),
        ],
        out_specs=pl.BlockSpec((be, HPW), lambda i: (i, 0)),
        out_shape=jax.ShapeDtypeStruct((e, HPW), jnp.float32),
    )(hpr, hpc, wp['w1a'], wp['w1b'], wp['w1d'], wp['b1'],
      wp['w2'], wp['b2'], wp['wc1'], wp['bc1'], wp['wc2t'])


def _node_body(hp_ref, aggp_ref,
               wn1a, wn1b, bn1, wn2, bn2, g, b, mask3, cnt_sel,
               out_ref):
    hp = hp_ref[...]
    h = hp[:, :HID]
    pos = hp[:, HID:]
    acc = aggp_ref[0] + aggp_ref[1]
    agg = acc[:, :HID]
    cuc = acc[:, HID:]
    nu = _silu(h @ wn1a[...] + agg @ wn1b[...] + bn1[...])
    nu = nu @ wn2[...] + bn2[...]
    x = h + nu
    mu = jnp.mean(x, axis=1, keepdims=True)
    xc = x - mu
    var = jnp.mean(xc * xc, axis=1, keepdims=True)
    hn = xc * lax.rsqrt(var + 1e-5) * g[...] + b[...]
    cnt = jnp.sum(cuc * cnt_sel[...], axis=1, keepdims=True)
    posn = pos + cuc * mask3[...] / (cnt + 1e-6)
    out_ref[...] = jnp.concatenate([hn, posn], axis=1)


def _node_call(hp, aggp, wp, mask3, cnt_sel, bn):
    n = hp.shape[0]
    grid = (n // bn,)
    full = lambda shape: pl.BlockSpec(shape, lambda i: (0, 0))
    return pl.pallas_call(
        _node_body,
        grid=grid,
        in_specs=[
            pl.BlockSpec((bn, HPW), lambda i: (i, 0)),
            pl.BlockSpec((NC, bn, HPW), lambda i: (0, i, 0)),
            full((HID, HID)), full((HID, HID)), full((1, HID)),
            full((HID, HID)), full((1, HID)),
            full((1, HID)), full((1, HID)),
            full((1, POSW)), full((1, POSW)),
        ],
        out_specs=pl.BlockSpec((bn, HPW), lambda i: (i, 0)),
        out_shape=jax.ShapeDtypeStruct((n, HPW), jnp.float32),
    )(hp, aggp, wp['wn1a'], wp['wn1b'], wp['bn1'],
      wp['wn2'], wp['bn2'], wp['g'], wp['b'], mask3, cnt_sel)


def _pool_body(hp_ref, bids_ref, wo0, bo0, wo1, bo1, wo2, bo2,
               out_ref, sums, cnts):
    i = pl.program_id(0)
    nblk = pl.num_programs(0)

    @pl.when(i == 0)
    def _():
        sums[...] = jnp.zeros_like(sums)
        cnts[...] = jnp.zeros_like(cnts)

    bn = hp_ref.shape[0]
    bp = sums.shape[0]
    bids = bids_ref[...].reshape(1, bn)
    oh = (lax.broadcasted_iota(jnp.int32, (bp, bn), 0) == bids).astype(jnp.float32)
    sums[...] += oh @ hp_ref[:, :HID]
    cnts[...] += jnp.sum(oh, axis=1, keepdims=True)

    @pl.when(i == nblk - 1)
    def _():
        gf = sums[...] / jnp.maximum(cnts[...], 1.0)
        gg = _silu(gf @ wo0[...] + bo0[...])
        gg = _silu(gg @ wo1[...] + bo1[...])
        out_ref[...] = gg @ wo2[...] + bo2[...]


def _pool_call(hp, bids3, wp, bp, bn):
    n = hp.shape[0]
    grid = (n // bn,)
    hh = HID // 2
    full = lambda shape: pl.BlockSpec(shape, lambda i: (0, 0))
    return pl.pallas_call(
        _pool_body,
        grid=grid,
        in_specs=[
            pl.BlockSpec((bn, HPW), lambda i: (i, 0)),
            pl.BlockSpec((1, 1, bn), lambda i: (i, 0, 0)),
            full((HID, HID)), full((1, HID)),
            full((HID, hh)), full((1, hh)),
            full((hh, HID)), full((1, HID)),
        ],
        out_specs=pl.BlockSpec((bp, HID), lambda i: (0, 0)),
        out_shape=jax.ShapeDtypeStruct((bp, HID), jnp.float32),
        scratch_shapes=[
            pltpu.VMEM((bp, HID), jnp.float32),
            pltpu.VMEM((bp, 1), jnp.float32),
        ],
    )(hp, bids3, wp['wo0'], wp['bo0'], wp['wo1'], wp['bo1'], wp['wo2'], wp['bo2'])


def kernel(pos, atom_types, params, edge_index, batch):
    n = pos.shape[0]
    e = edge_index.shape[1]
    b = 200
    lat = 64
    bn = 1000
    be = 2000
    bp = 256
    nb = e // CH

    row1 = edge_index[0]
    rc = jnp.stack([edge_index[0].reshape(nb, CH),
                    edge_index[1].reshape(nb, CH)], axis=1)
    pos16 = jnp.zeros((n, POSW), jnp.float32).at[:, :3].set(pos)
    zeros_p = jnp.zeros((n, HPW), jnp.float32)
    bids3 = batch.astype(jnp.int32).reshape(n // bn, 1, bn)

    r2 = lambda v: v.reshape(1, -1)
    emb = params['emb']
    embp = {'w0': emb[0]['W'], 'b0': r2(emb[0]['b']),
            'w1': emb[1]['W'], 'b1': r2(emb[1]['b']),
            'w2': emb[2]['W'], 'b2': r2(emb[2]['b'])}
    hp = _emb_call(atom_types, pos16, embp, bn)

    iota16 = jnp.arange(POSW)
    mask3 = (iota16 < 3).astype(jnp.float32).reshape(1, POSW)
    cnt_sel = (iota16 == 3).astype(jnp.float32).reshape(1, POSW)

    bf = lambda v: v.astype(jnp.bfloat16)
    for p in params['layers']:
        e1w = p['e1']['W']
        wp_e = {'w1a': bf(e1w[:HID]), 'w1b': bf(e1w[HID:2 * HID]),
                'w1d': e1w[2 * HID:2 * HID + 1], 'b1': r2(p['e1']['b']),
                'w2': bf(p['e2']['W']), 'b2': r2(p['e2']['b']),
                'wc1': bf(p['c1']['W']), 'bc1': r2(p['c1']['b']),
                'wc2t': p['c2']['W'].reshape(1, HID)}
        n1w = p['n1']['W']
        wp_n = {'wn1a': n1w[:HID], 'wn1b': n1w[HID:],
                'bn1': r2(p['n1']['b']), 'wn2': p['n2']['W'],
                'bn2': r2(p['n2']['b']), 'g': r2(p['ln_g']), 'b': r2(p['ln_b'])}

        hpr, hpc = _sc_gather(hp, rc)
        pay = _edge_call(hpr, hpc, wp_e, be)
        aggp = _sc_scatter(pay, row1, zeros_p)
        hp = _node_call(hp, aggp, wp_n, mask3, cnt_sel, bn)

    out = params['out']
    wp_o = {'wo0': out[0]['W'], 'bo0': r2(out[0]['b']),
            'wo1': out[1]['W'], 'bo1': r2(out[1]['b']),
            'wo2': out[2]['W'], 'bo2': r2(out[2]['b'])}
    lp = _pool_call(hp, bids3, wp_o, bp, bn)
    return lp[:b, :lat], lp[:b, lat:2 * lat]


# speedup vs baseline: 1.7718x; 1.7718x over previous
import functools

import jax
import jax.numpy as jnp
from jax import lax
from jax.experimental import pallas as pl
from jax.experimental.pallas import tpu as pltpu
from jax.experimental.pallas import tpu_sc as plsc

HID = 128
POSW = 16
CWW = 16
CH = 128
NC = 2
NS = 16
NW = NC * NS


def _silu(x):
    return x * jax.nn.sigmoid(x)


def _sc_gather(h, pos16, rc):
    n = h.shape[0]
    nb = rc.shape[0]
    e = nb * CH
    base_ct = nb // NW
    rem = nb - base_ct * NW
    npair = base_ct // 2
    mesh = plsc.VectorSubcoreMesh(core_axis_name="c", subcore_axis_name="s",
                                  num_cores=NC, num_subcores=NS)

    def body(h_hbm, pos_hbm, rc_hbm,
             hr_hbm, hc_hbm, pr_hbm, pc_hbm,
             idx_v0, idx_v1, hr_v0, hr_v1, hc_v0, hc_v1,
             pr_v0, pr_v1, pc_v0, pc_v1, semg, sems):
        cid = lax.axis_index("c")
        sid = lax.axis_index("s")
        wid = sid * NC + cid
        idx_v = (idx_v0, idx_v1)
        hr_v = (hr_v0, hr_v1)
        hc_v = (hc_v0, hc_v1)
        pr_v = (pr_v0, pr_v1)
        pc_v = (pc_v0, pc_v1)

        def fire(bs, k):
            r = k * NW + wid
            pltpu.sync_copy(rc_hbm.at[r], idx_v[bs])
            pltpu.async_copy(h_hbm.at[idx_v[bs].at[0]], hr_v[bs], semg)
            pltpu.async_copy(h_hbm.at[idx_v[bs].at[1]], hc_v[bs], semg)
            pltpu.async_copy(pos_hbm.at[idx_v[bs].at[0]], pr_v[bs], semg)
            pltpu.async_copy(pos_hbm.at[idx_v[bs].at[1]], pc_v[bs], semg)

        def wait_gathers(bs):
            pltpu.make_async_copy(h_hbm.at[idx_v[bs].at[0]], hr_v[bs], semg).wait()
            pltpu.make_async_copy(h_hbm.at[idx_v[bs].at[1]], hc_v[bs], semg).wait()
            pltpu.make_async_copy(pos_hbm.at[idx_v[bs].at[0]], pr_v[bs], semg).wait()
            pltpu.make_async_copy(pos_hbm.at[idx_v[bs].at[1]], pc_v[bs], semg).wait()

        def store(bs, k):
            base = (k * NW + wid) * CH
            pltpu.async_copy(hr_v[bs], hr_hbm.at[pl.ds(base, CH)], sems)
            pltpu.async_copy(hc_v[bs], hc_hbm.at[pl.ds(base, CH)], sems)
            pltpu.async_copy(pr_v[bs], pr_hbm.at[pl.ds(base, CH)], sems)
            pltpu.async_copy(pc_v[bs], pc_hbm.at[pl.ds(base, CH)], sems)

        def wait_stores(bs):
            pltpu.make_async_copy(hr_v[bs], hr_hbm.at[pl.ds(0, CH)], sems).wait()
            pltpu.make_async_copy(hc_v[bs], hc_hbm.at[pl.ds(0, CH)], sems).wait()
            pltpu.make_async_copy(pr_v[bs], pr_hbm.at[pl.ds(0, CH)], sems).wait()
            pltpu.make_async_copy(pc_v[bs], pc_hbm.at[pl.ds(0, CH)], sems).wait()

        fire(0, 0)

        def pair(g, carry):
            c0 = 2 * g

            @pl.when(g > 0)
            def _():
                wait_stores(1)
            fire(1, c0 + 1)
            wait_gathers(0)
            store(0, c0)
            wait_gathers(1)

            @pl.when(g < npair - 1)
            def _():
                wait_stores(0)
                fire(0, c0 + 2)
            store(1, c0 + 1)
            return carry

        lax.fori_loop(0, npair, pair, 0)

        if rem:
            @pl.when(wid < rem)
            def _():
                wait_stores(0)
                fire(0, base_ct)
                wait_gathers(0)
                store(0, base_ct)
        wait_stores(0)
        wait_stores(1)

    f = pl.kernel(
        body,
        out_type=[
            jax.ShapeDtypeStruct((e, HID), jnp.float32),
            jax.ShapeDtypeStruct((e, HID), jnp.float32),
            jax.ShapeDtypeStruct((e, POSW), jnp.float32),
            jax.ShapeDtypeStruct((e, POSW), jnp.float32),
        ],
        mesh=mesh,
        compiler_params=pltpu.CompilerParams(use_tc_tiling_on_sc=False),
        scratch_types=[
            pltpu.VMEM((2, CH), jnp.int32),
            pltpu.VMEM((2, CH), jnp.int32),
            pltpu.VMEM((CH, HID), jnp.float32),
            pltpu.VMEM((CH, HID), jnp.float32),
            pltpu.VMEM((CH, HID), jnp.float32),
            pltpu.VMEM((CH, HID), jnp.float32),
            pltpu.VMEM((CH, POSW), jnp.float32),
            pltpu.VMEM((CH, POSW), jnp.float32),
            pltpu.VMEM((CH, POSW), jnp.float32),
            pltpu.VMEM((CH, POSW), jnp.float32),
            pltpu.SemaphoreType.DMA,
            pltpu.SemaphoreType.DMA,
        ],
    )
    return f(h, pos16, rc)


def _sc_scatter(m, cwx, row1, zeros_m, zeros_c):
    n = zeros_m.shape[0]
    e = row1.shape[0]
    nb = e // CH
    base_ct = nb // NW
    rem = nb - base_ct * NW
    npair = base_ct // 2
    rpt = (n // NS) // 8 * 8
    ex = n - rpt * NS
    mesh = plsc.VectorSubcoreMesh(core_axis_name="c", subcore_axis_name="s",
                                  num_cores=NC, num_subcores=NS)

    def body(m_hbm, cwx_hbm, row_hbm, zm_hbm, zc_hbm,
             aggp_hbm, cwp_hbm,
             idx_v0, idx_v1, m_v0, m_v1, c_v0, c_v1,
             agg_sh, cw_sh, seml, sema):
        cid = lax.axis_index("c")
        sid = lax.axis_index("s")
        wid = sid * NC + cid
        r0 = sid * rpt
        idx_v = (idx_v0, idx_v1)
        m_v = (m_v0, m_v1)
        c_v = (c_v0, c_v1)

        pltpu.sync_copy(zm_hbm.at[pl.ds(r0, rpt)], agg_sh.at[pl.ds(r0, rpt)])
        pltpu.sync_copy(zc_hbm.at[pl.ds(r0, rpt)], cw_sh.at[pl.ds(r0, rpt)])
        if ex:
            @pl.when(sid == 0)
            def _():
                pltpu.sync_copy(zm_hbm.at[pl.ds(rpt * NS, ex)],
                                agg_sh.at[pl.ds(rpt * NS, ex)])
                pltpu.sync_copy(zc_hbm.at[pl.ds(rpt * NS, ex)],
                                cw_sh.at[pl.ds(rpt * NS, ex)])
        plsc.subcore_barrier()

        def load(bs, k):
            base = (k * NW + wid) * CH
            pltpu.async_copy(row_hbm.at[pl.ds(base, CH)], idx_v[bs], seml)
            pltpu.async_copy(m_hbm.at[pl.ds(base, CH)], m_v[bs], seml)
            pltpu.async_copy(cwx_hbm.at[pl.ds(base, CH)], c_v[bs], seml)

        def wait_loads(bs):
            pltpu.make_async_copy(row_hbm.at[pl.ds(0, CH)], idx_v[bs], seml).wait()
            pltpu.make_async_copy(m_hbm.at[pl.ds(0, CH)], m_v[bs], seml).wait()
            pltpu.make_async_copy(cwx_hbm.at[pl.ds(0, CH)], c_v[bs], seml).wait()

        def scat(bs):
            pltpu.async_copy(m_v[bs], agg_sh.at[idx_v[bs]], sema, add=True)
            pltpu.async_copy(c_v[bs], cw_sh.at[idx_v[bs]], sema, add=True)

        def wait_scat(bs):
            pltpu.make_async_copy(m_v[bs], agg_sh.at[idx_v[bs]], sema).wait()
            pltpu.make_async_copy(c_v[bs], cw_sh.at[idx_v[bs]], sema).wait()

        load(0, 0)

        def pair(g, carry):
            c0 = 2 * g

            @pl.when(g > 0)
            def _():
                wait_scat(1)
            load(1, c0 + 1)
            wait_loads(0)
            scat(0)
            wait_loads(1)

            @pl.when(g < npair - 1)
            def _():
                wait_scat(0)
                load(0, c0 + 2)
            scat(1)
            return carry

        lax.fori_loop(0, npair, pair, 0)

        if rem:
            @pl.when(wid < rem)
            def _():
                wait_scat(0)
                load(0, base_ct)
                wait_loads(0)
                scat(0)
        wait_scat(0)
        wait_scat(1)
        plsc.subcore_barrier()

        pltpu.sync_copy(agg_sh.at[pl.ds(r0, rpt)],
                        aggp_hbm.at[cid, pl.ds(r0, rpt)])
        pltpu.sync_copy(cw_sh.at[pl.ds(r0, rpt)],
                        cwp_hbm.at[cid, pl.ds(r0, rpt)])
        if ex:
            @pl.when(sid == 0)
            def _():
                pltpu.sync_copy(agg_sh.at[pl.ds(rpt * NS, ex)],
                                aggp_hbm.at[cid, pl.ds(rpt * NS, ex)])
                pltpu.sync_copy(cw_sh.at[pl.ds(rpt * NS, ex)],
                                cwp_hbm.at[cid, pl.ds(rpt * NS, ex)])

    f = pl.kernel(
        body,
        out_type=[
            jax.ShapeDtypeStruct((NC, n, HID), jnp.float32),
            jax.ShapeDtypeStruct((NC, n, CWW), jnp.float32),
        ],
        mesh=mesh,
        compiler_params=pltpu.CompilerParams(use_tc_tiling_on_sc=False),
        scratch_types=[
            pltpu.VMEM((CH,), jnp.int32),
            pltpu.VMEM((CH,), jnp.int32),
            pltpu.VMEM((CH, HID), jnp.float32),
            pltpu.VMEM((CH, HID), jnp.float32),
            pltpu.VMEM((CH, CWW), jnp.float32),
            pltpu.VMEM((CH, CWW), jnp.float32),
            pltpu.VMEM_SHARED((n, HID), jnp.float32),
            pltpu.VMEM_SHARED((n, CWW), jnp.float32),
            pltpu.SemaphoreType.DMA,
            pltpu.SemaphoreType.DMA,
        ],
    )
    return f(m, cwx, row1, zeros_m, zeros_c)


def _emb_body(x_ref, w0, b0, w1, b1, w2, b2, out_ref):
    h = _silu(x_ref[...] @ w0[...] + b0[...])
    h = _silu(h @ w1[...] + b1[...])
    out_ref[...] = h @ w2[...] + b2[...]


def _emb_call(x, p, bn):
    n, af = x.shape
    grid = (n // bn,)
    full = lambda shape: pl.BlockSpec(shape, lambda i: (0, 0))
    return pl.pallas_call(
        _emb_body,
        grid=grid,
        in_specs=[
            pl.BlockSpec((bn, af), lambda i: (i, 0)),
            full((af, HID)), full((1, HID)),
            full((HID, HID)), full((1, HID)),
            full((HID, HID)), full((1, HID)),
        ],
        out_specs=pl.BlockSpec((bn, HID), lambda i: (i, 0)),
        out_shape=jax.ShapeDtypeStruct((n, HID), jnp.float32),
    )(x, p['w0'], p['b0'], p['w1'], p['b1'], p['w2'], p['b2'])


def _edge_body(hr_ref, hc_ref, pr_ref, pc_ref,
               w1a, w1b, w1d, b1, w2, b2, wc1, bc1, wc2t,
               m_ref, cwx_ref):
    dot = lambda a, w: lax.dot(a, w, preferred_element_type=jnp.float32)
    rel = pr_ref[...] - pc_ref[...]
    d2 = jnp.sum(rel * rel, axis=1, keepdims=True)
    t = dot(hr_ref[...].astype(jnp.bfloat16), w1a[...]) \
        + dot(hc_ref[...].astype(jnp.bfloat16), w1b[...]) \
        + d2 * w1d[...] + b1[...]
    m1 = _silu(t).astype(jnp.bfloat16)
    m = _silu(dot(m1, w2[...]) + b2[...])
    c1 = _silu(dot(m.astype(jnp.bfloat16), wc1[...]) + bc1[...])
    cw = jnp.sum(c1 * wc2t[...], axis=1, keepdims=True)
    be = rel.shape[0]
    cnt1 = (lax.broadcasted_iota(jnp.int32, (be, CWW), 1) == 3).astype(jnp.float32)
    m_ref[...] = m
    cwx_ref[...] = cw * rel + cnt1


def _edge_call(hr, hc, pr, pc, wp, be):
    e = hr.shape[0]
    grid = (e // be,)
    full = lambda shape: pl.BlockSpec(shape, lambda i: (0, 0))
    return pl.pallas_call(
        _edge_body,
        grid=grid,
        in_specs=[
            pl.BlockSpec((be, HID), lambda i: (i, 0)),
            pl.BlockSpec((be, HID), lambda i: (i, 0)),
            pl.BlockSpec((be, POSW), lambda i: (i, 0)),
            pl.BlockSpec((be, POSW), lambda i: (i, 0)),
            full((HID, HID)), full((HID, HID)), full((1, HID)), full((1, HID)),
            full((HID, HID)), full((1, HID)),
            full((HID, HID)), full((1, HID)), full((1, HID)),
        ],
        out_specs=[
            pl.BlockSpec((be, HID), lambda i: (i, 0)),
            pl.BlockSpec((be, CWW), lambda i: (i, 0)),
        ],
        out_shape=[
            jax.ShapeDtypeStruct((e, HID), jnp.float32),
            jax.ShapeDtypeStruct((e, CWW), jnp.float32),
        ],
    )(hr, hc, pr, pc, wp['w1a'], wp['w1b'], wp['w1d'], wp['b1'],
      wp['w2'], wp['b2'], wp['wc1'], wp['bc1'], wp['wc2t'])


def _node_body(h_ref, aggp_ref, cwp_ref, pos_ref,
               wn1a, wn1b, bn1, wn2, bn2, g, b, mask3, cnt_sel,
               hout_ref, posout_ref):
    h = h_ref[...]
    agg = aggp_ref[0] + aggp_ref[1]
    cuc = cwp_ref[0] + cwp_ref[1]
    nu = _silu(h @ wn1a[...] + agg @ wn1b[...] + bn1[...])
    nu = nu @ wn2[...] + bn2[...]
    x = h + nu
    mu = jnp.mean(x, axis=1, keepdims=True)
    xc = x - mu
    var = jnp.mean(xc * xc, axis=1, keepdims=True)
    hout_ref[...] = xc * lax.rsqrt(var + 1e-5) * g[...] + b[...]
    cnt = jnp.sum(cuc * cnt_sel[...], axis=1, keepdims=True)
    posout_ref[...] = pos_ref[...] + cuc * mask3[...] / (cnt + 1e-6)


def _node_call(h, aggp, cwp, pos16, wp, mask3, cnt_sel, bn):
    n = h.shape[0]
    grid = (n // bn,)
    full = lambda shape: pl.BlockSpec(shape, lambda i: (0, 0))
    return pl.pallas_call(
        _node_body,
        grid=grid,
        in_specs=[
            pl.BlockSpec((bn, HID), lambda i: (i, 0)),
            pl.BlockSpec((NC, bn, HID), lambda i: (0, i, 0)),
            pl.BlockSpec((NC, bn, CWW), lambda i: (0, i, 0)),
            pl.BlockSpec((bn, POSW), lambda i: (i, 0)),
            full((HID, HID)), full((HID, HID)), full((1, HID)),
            full((HID, HID)), full((1, HID)),
            full((1, HID)), full((1, HID)),
            full((1, CWW)), full((1, CWW)),
        ],
        out_specs=[
            pl.BlockSpec((bn, HID), lambda i: (i, 0)),
            pl.BlockSpec((bn, POSW), lambda i: (i, 0)),
        ],
        out_shape=[
            jax.ShapeDtypeStruct((n, HID), jnp.float32),
            jax.ShapeDtypeStruct((n, POSW), jnp.float32),
        ],
    )(h, aggp, cwp, pos16, wp['wn1a'], wp['wn1b'], wp['bn1'],
      wp['wn2'], wp['bn2'], wp['g'], wp['b'], mask3, cnt_sel)


def _pool_body(h_ref, bids_ref, wo0, bo0, wo1, bo1, wo2, bo2,
               out_ref, sums, cnts):
    i = pl.program_id(0)
    nblk = pl.num_programs(0)

    @pl.when(i == 0)
    def _():
        sums[...] = jnp.zeros_like(sums)
        cnts[...] = jnp.zeros_like(cnts)

    bn = h_ref.shape[0]
    bp = sums.shape[0]
    bids = bids_ref[...].reshape(1, bn)
    oh = (lax.broadcasted_iota(jnp.int32, (bp, bn), 0) == bids).astype(jnp.float32)
    sums[...] += oh @ h_ref[...]
    cnts[...] += jnp.sum(oh, axis=1, keepdims=True)

    @pl.when(i == nblk - 1)
    def _():
        gf = sums[...] / jnp.maximum(cnts[...], 1.0)
        gg = _silu(gf @ wo0[...] + bo0[...])
        gg = _silu(gg @ wo1[...] + bo1[...])
        out_ref[...] = gg @ wo2[...] + bo2[...]


def _pool_call(h, bids3, wp, bp, bn):
    n = h.shape[0]
    grid = (n // bn,)
    hh = HID // 2
    full = lambda shape: pl.BlockSpec(shape, lambda i: (0, 0))
    return pl.pallas_call(
        _pool_body,
        grid=grid,
        in_specs=[
            pl.BlockSpec((bn, HID), lambda i: (i, 0)),
            pl.BlockSpec((1, 1, bn), lambda i: (i, 0, 0)),
            full((HID, HID)), full((1, HID)),
            full((HID, hh)), full((1, hh)),
            full((hh, HID)), full((1, HID)),
        ],
        out_specs=pl.BlockSpec((bp, HID), lambda i: (0, 0)),
        out_shape=jax.ShapeDtypeStruct((bp, HID), jnp.float32),
        scratch_shapes=[
            pltpu.VMEM((bp, HID), jnp.float32),
            pltpu.VMEM((bp, 1), jnp.float32),
        ],
    )(h, bids3, wp['wo0'], wp['bo0'], wp['wo1'], wp['bo1'], wp['wo2'], wp['bo2'])


def kernel(pos, atom_types, params, edge_index, batch):
    n = pos.shape[0]
    e = edge_index.shape[1]
    b = 200
    lat = 64
    bn = 1000
    be = 2000
    bp = 256
    nb = e // CH

    row1 = edge_index[0]
    rc = jnp.stack([edge_index[0].reshape(nb, CH),
                    edge_index[1].reshape(nb, CH)], axis=1)
    pos16 = jnp.zeros((n, POSW), jnp.float32).at[:, :3].set(pos)
    zeros_m = jnp.zeros((n, HID), jnp.float32)
    zeros_c = jnp.zeros((n, CWW), jnp.float32)
    bids3 = batch.astype(jnp.int32).reshape(n // bn, 1, bn)

    r2 = lambda v: v.reshape(1, -1)
    emb = params['emb']
    embp = {'w0': emb[0]['W'], 'b0': r2(emb[0]['b']),
            'w1': emb[1]['W'], 'b1': r2(emb[1]['b']),
            'w2': emb[2]['W'], 'b2': r2(emb[2]['b'])}
    h = _emb_call(atom_types, embp, bn)

    iota16 = jnp.arange(CWW)
    mask3 = (iota16 < 3).astype(jnp.float32).reshape(1, CWW)
    cnt_sel = (iota16 == 3).astype(jnp.float32).reshape(1, CWW)

    bf = lambda v: v.astype(jnp.bfloat16)
    for p in params['layers']:
        e1w = p['e1']['W']
        wp_e = {'w1a': bf(e1w[:HID]), 'w1b': bf(e1w[HID:2 * HID]),
                'w1d': e1w[2 * HID:2 * HID + 1], 'b1': r2(p['e1']['b']),
                'w2': bf(p['e2']['W']), 'b2': r2(p['e2']['b']),
                'wc1': bf(p['c1']['W']), 'bc1': r2(p['c1']['b']),
                'wc2t': p['c2']['W'].reshape(1, HID)}
        n1w = p['n1']['W']
        wp_n = {'wn1a': n1w[:HID], 'wn1b': n1w[HID:],
                'bn1': r2(p['n1']['b']), 'wn2': p['n2']['W'],
                'bn2': r2(p['n2']['b']), 'g': r2(p['ln_g']), 'b': r2(p['ln_b'])}

        hr, hc, pr, pc = _sc_gather(h, pos16, rc)
        m, cwx = _edge_call(hr, hc, pr, pc, wp_e, be)
        aggp, cwp = _sc_scatter(m, cwx, row1, zeros_m, zeros_c)
        h, pos16 = _node_call(h, aggp, cwp, pos16, wp_n, mask3, cnt_sel, bn)

    out = params['out']
    wp_o = {'wo0': out[0]['W'], 'bo0': r2(out[0]['b']),
            'wo1': out[1]['W'], 'bo1': r2(out[1]['b']),
            'wo2': out[2]['W'], 'bo2': r2(out[2]['b'])}
    lp = _pool_call(h, bids3, wp_o, bp, bn)
    return lp[:b, :lat], lp[:b, lat:2 * lat]


# speedup vs baseline: 1.8691x; 1.0550x over previous
import functools

import jax
import jax.numpy as jnp
from jax import lax
from jax.experimental import pallas as pl
from jax.experimental.pallas import tpu as pltpu
from jax.experimental.pallas import tpu_sc as plsc

HID = 128
POSW = 16
CWW = 16
CH = 128
NC = 2
NS = 16
NW = NC * NS


def _silu(x):
    return x * jax.nn.sigmoid(x)


def _sc_gather(h, pos16, rc):
    n = h.shape[0]
    nb = rc.shape[0]
    e = nb * CH
    base_ct = nb // NW
    rem = nb - base_ct * NW
    npair = base_ct // 2
    mesh = plsc.VectorSubcoreMesh(core_axis_name="c", subcore_axis_name="s",
                                  num_cores=NC, num_subcores=NS)

    def body(h_hbm, pos_hbm, rc_hbm,
             hr_hbm, hc_hbm, pr_hbm, pc_hbm,
             idx_v0, idx_v1, hr_v0, hr_v1, hc_v0, hc_v1,
             pr_v0, pr_v1, pc_v0, pc_v1, semg, sems):
        cid = lax.axis_index("c")
        sid = lax.axis_index("s")
        wid = sid * NC + cid
        idx_v = (idx_v0, idx_v1)
        hr_v = (hr_v0, hr_v1)
        hc_v = (hc_v0, hc_v1)
        pr_v = (pr_v0, pr_v1)
        pc_v = (pc_v0, pc_v1)

        def fire(bs, k):
            r = k * NW + wid
            pltpu.sync_copy(rc_hbm.at[r], idx_v[bs])
            pltpu.async_copy(h_hbm.at[idx_v[bs].at[0]], hr_v[bs], semg)
            pltpu.async_copy(h_hbm.at[idx_v[bs].at[1]], hc_v[bs], semg)
            pltpu.async_copy(pos_hbm.at[idx_v[bs].at[0]], pr_v[bs], semg)
            pltpu.async_copy(pos_hbm.at[idx_v[bs].at[1]], pc_v[bs], semg)

        def wait_gathers(bs):
            pltpu.make_async_copy(h_hbm.at[idx_v[bs].at[0]], hr_v[bs], semg).wait()
            pltpu.make_async_copy(h_hbm.at[idx_v[bs].at[1]], hc_v[bs], semg).wait()
            pltpu.make_async_copy(pos_hbm.at[idx_v[bs].at[0]], pr_v[bs], semg).wait()
            pltpu.make_async_copy(pos_hbm.at[idx_v[bs].at[1]], pc_v[bs], semg).wait()

        def store(bs, k):
            base = (k * NW + wid) * CH
            pltpu.async_copy(hr_v[bs], hr_hbm.at[pl.ds(base, CH)], sems)
            pltpu.async_copy(hc_v[bs], hc_hbm.at[pl.ds(base, CH)], sems)
            pltpu.async_copy(pr_v[bs], pr_hbm.at[pl.ds(base, CH)], sems)
            pltpu.async_copy(pc_v[bs], pc_hbm.at[pl.ds(base, CH)], sems)

        def wait_stores(bs):
            pltpu.make_async_copy(hr_v[bs], hr_hbm.at[pl.ds(0, CH)], sems).wait()
            pltpu.make_async_copy(hc_v[bs], hc_hbm.at[pl.ds(0, CH)], sems).wait()
            pltpu.make_async_copy(pr_v[bs], pr_hbm.at[pl.ds(0, CH)], sems).wait()
            pltpu.make_async_copy(pc_v[bs], pc_hbm.at[pl.ds(0, CH)], sems).wait()

        fire(0, 0)

        def pair(g, carry):
            c0 = 2 * g

            @pl.when(g > 0)
            def _():
                wait_stores(1)
            fire(1, c0 + 1)
            wait_gathers(0)
            store(0, c0)
            wait_gathers(1)

            @pl.when(g < npair - 1)
            def _():
                wait_stores(0)
                fire(0, c0 + 2)
            store(1, c0 + 1)
            return carry

        lax.fori_loop(0, npair, pair, 0)

        if rem:
            @pl.when(wid < rem)
            def _():
                wait_stores(0)
                fire(0, base_ct)
                wait_gathers(0)
                store(0, base_ct)
        wait_stores(0)
        wait_stores(1)

    f = pl.kernel(
        body,
        out_type=[
            jax.ShapeDtypeStruct((e, HID), jnp.float32),
            jax.ShapeDtypeStruct((e, HID), jnp.float32),
            jax.ShapeDtypeStruct((e, POSW), jnp.float32),
            jax.ShapeDtypeStruct((e, POSW), jnp.float32),
        ],
        mesh=mesh,
        compiler_params=pltpu.CompilerParams(use_tc_tiling_on_sc=False),
        scratch_types=[
            pltpu.VMEM((2, CH), jnp.int32),
            pltpu.VMEM((2, CH), jnp.int32),
            pltpu.VMEM((CH, HID), jnp.float32),
            pltpu.VMEM((CH, HID), jnp.float32),
            pltpu.VMEM((CH, HID), jnp.float32),
            pltpu.VMEM((CH, HID), jnp.float32),
            pltpu.VMEM((CH, POSW), jnp.float32),
            pltpu.VMEM((CH, POSW), jnp.float32),
            pltpu.VMEM((CH, POSW), jnp.float32),
            pltpu.VMEM((CH, POSW), jnp.float32),
            pltpu.SemaphoreType.DMA,
            pltpu.SemaphoreType.DMA,
        ],
    )
    return f(h, pos16, rc)


def _sc_scatter(m, cwx, row1, zeros_m, zeros_c):
    n = zeros_m.shape[0]
    e = row1.shape[0]
    nb = e // CH
    base_ct = nb // NW
    rem = nb - base_ct * NW
    npair = base_ct // 2
    rpt = (n // NS) // 8 * 8
    ex = n - rpt * NS
    mesh = plsc.VectorSubcoreMesh(core_axis_name="c", subcore_axis_name="s",
                                  num_cores=NC, num_subcores=NS)

    def body(m_hbm, cwx_hbm, row_hbm, zm_hbm, zc_hbm,
             aggp_hbm, cwp_hbm,
             idx_v0, idx_v1, m_v0, m_v1, c_v0, c_v1,
             agg_sh, cw_sh, seml, sema):
        cid = lax.axis_index("c")
        sid = lax.axis_index("s")
        wid = sid * NC + cid
        r0 = sid * rpt
        idx_v = (idx_v0, idx_v1)
        m_v = (m_v0, m_v1)
        c_v = (c_v0, c_v1)

        pltpu.sync_copy(zm_hbm.at[pl.ds(r0, rpt)], agg_sh.at[pl.ds(r0, rpt)])
        pltpu.sync_copy(zc_hbm.at[pl.ds(r0, rpt)], cw_sh.at[pl.ds(r0, rpt)])
        if ex:
            @pl.when(sid == 0)
            def _():
                pltpu.sync_copy(zm_hbm.at[pl.ds(rpt * NS, ex)],
                                agg_sh.at[pl.ds(rpt * NS, ex)])
                pltpu.sync_copy(zc_hbm.at[pl.ds(rpt * NS, ex)],
                                cw_sh.at[pl.ds(rpt * NS, ex)])
        plsc.subcore_barrier()

        def load(bs, k):
            base = (k * NW + wid) * CH
            pltpu.async_copy(row_hbm.at[pl.ds(base, CH)], idx_v[bs], seml)
            pltpu.async_copy(m_hbm.at[pl.ds(base, CH)], m_v[bs], seml)
            pltpu.async_copy(cwx_hbm.at[pl.ds(base, CH)], c_v[bs], seml)

        def wait_loads(bs):
            pltpu.make_async_copy(row_hbm.at[pl.ds(0, CH)], idx_v[bs], seml).wait()
            pltpu.make_async_copy(m_hbm.at[pl.ds(0, CH)], m_v[bs], seml).wait()
            pltpu.make_async_copy(cwx_hbm.at[pl.ds(0, CH)], c_v[bs], seml).wait()

        def scat(bs):
            pltpu.async_copy(m_v[bs], agg_sh.at[idx_v[bs]], sema, add=True)
            pltpu.async_copy(c_v[bs], cw_sh.at[idx_v[bs]], sema, add=True)

        def wait_scat(bs):
            pltpu.make_async_copy(m_v[bs], agg_sh.at[idx_v[bs]], sema).wait()
            pltpu.make_async_copy(c_v[bs], cw_sh.at[idx_v[bs]], sema).wait()

        load(0, 0)

        def pair(g, carry):
            c0 = 2 * g

            @pl.when(g > 0)
            def _():
                wait_scat(1)
            load(1, c0 + 1)
            wait_loads(0)
            scat(0)
            wait_loads(1)

            @pl.when(g < npair - 1)
            def _():
                wait_scat(0)
                load(0, c0 + 2)
            scat(1)
            return carry

        lax.fori_loop(0, npair, pair, 0)

        if rem:
            @pl.when(wid < rem)
            def _():
                wait_scat(0)
                load(0, base_ct)
                wait_loads(0)
                scat(0)
        wait_scat(0)
        wait_scat(1)
        plsc.subcore_barrier()

        pltpu.sync_copy(agg_sh.at[pl.ds(r0, rpt)],
                        aggp_hbm.at[cid, pl.ds(r0, rpt)])
        pltpu.sync_copy(cw_sh.at[pl.ds(r0, rpt)],
                        cwp_hbm.at[cid, pl.ds(r0, rpt)])
        if ex:
            @pl.when(sid == 0)
            def _():
                pltpu.sync_copy(agg_sh.at[pl.ds(rpt * NS, ex)],
                                aggp_hbm.at[cid, pl.ds(rpt * NS, ex)])
                pltpu.sync_copy(cw_sh.at[pl.ds(rpt * NS, ex)],
                                cwp_hbm.at[cid, pl.ds(rpt * NS, ex)])

    f = pl.kernel(
        body,
        out_type=[
            jax.ShapeDtypeStruct((NC, n, HID), jnp.float32),
            jax.ShapeDtypeStruct((NC, n, CWW), jnp.float32),
        ],
        mesh=mesh,
        compiler_params=pltpu.CompilerParams(use_tc_tiling_on_sc=False),
        scratch_types=[
            pltpu.VMEM((CH,), jnp.int32),
            pltpu.VMEM((CH,), jnp.int32),
            pltpu.VMEM((CH, HID), jnp.float32),
            pltpu.VMEM((CH, HID), jnp.float32),
            pltpu.VMEM((CH, CWW), jnp.float32),
            pltpu.VMEM((CH, CWW), jnp.float32),
            pltpu.VMEM_SHARED((n, HID), jnp.float32),
            pltpu.VMEM_SHARED((n, CWW), jnp.float32),
            pltpu.SemaphoreType.DMA,
            pltpu.SemaphoreType.DMA,
        ],
    )
    return f(m, cwx, row1, zeros_m, zeros_c)


def _emb_body(x_ref, w0, b0, w1, b1, w2, b2, out_ref):
    h = _silu(x_ref[...] @ w0[...] + b0[...])
    h = _silu(h @ w1[...] + b1[...])
    out_ref[...] = h @ w2[...] + b2[...]


def _emb_call(x, p, bn):
    n, af = x.shape
    grid = (n // bn,)
    full = lambda shape: pl.BlockSpec(shape, lambda i: (0, 0))
    return pl.pallas_call(
        _emb_body,
        grid=grid,
        in_specs=[
            pl.BlockSpec((bn, af), lambda i: (i, 0)),
            full((af, HID)), full((1, HID)),
            full((HID, HID)), full((1, HID)),
            full((HID, HID)), full((1, HID)),
        ],
        out_specs=pl.BlockSpec((bn, HID), lambda i: (i, 0)),
        out_shape=jax.ShapeDtypeStruct((n, HID), jnp.float32),
    )(x, p['w0'], p['b0'], p['w1'], p['b1'], p['w2'], p['b2'])


def _edge_body(hr_ref, hc_ref, pr_ref, pc_ref,
               w1a, w1b, w1d, b1, w2, b2, wc1, bc1, wc2t,
               m_ref, cwx_ref):
    dot = lambda a, w: lax.dot(a, w, preferred_element_type=jnp.float32)
    rel = pr_ref[...] - pc_ref[...]
    d2 = jnp.sum(rel * rel, axis=1, keepdims=True)
    t = dot(hr_ref[...].astype(jnp.bfloat16), w1a[...]) \
        + dot(hc_ref[...].astype(jnp.bfloat16), w1b[...]) \
        + d2 * w1d[...] + b1[...]
    m1 = _silu(t).astype(jnp.bfloat16)
    m = _silu(dot(m1, w2[...]) + b2[...])
    c1 = _silu(dot(m.astype(jnp.bfloat16), wc1[...]) + bc1[...])
    cw = jnp.sum(c1 * wc2t[...], axis=1, keepdims=True)
    be = rel.shape[0]
    cnt1 = (lax.broadcasted_iota(jnp.int32, (be, CWW), 1) == 3).astype(jnp.float32)
    m_ref[...] = m
    cwx_ref[...] = cw * rel + cnt1


def _edge_call(hr, hc, pr, pc, wp, be):
    e = hr.shape[0]
    grid = (e // be,)
    full = lambda shape: pl.BlockSpec(shape, lambda i: (0, 0))
    return pl.pallas_call(
        _edge_body,
        grid=grid,
        in_specs=[
            pl.BlockSpec((be, HID), lambda i: (i, 0)),
            pl.BlockSpec((be, HID), lambda i: (i, 0)),
            pl.BlockSpec((be, POSW), lambda i: (i, 0)),
            pl.BlockSpec((be, POSW), lambda i: (i, 0)),
            full((HID, HID)), full((HID, HID)), full((1, HID)), full((1, HID)),
            full((HID, HID)), full((1, HID)),
            full((HID, HID)), full((1, HID)), full((1, HID)),
        ],
        out_specs=[
            pl.BlockSpec((be, HID), lambda i: (i, 0)),
            pl.BlockSpec((be, CWW), lambda i: (i, 0)),
        ],
        out_shape=[
            jax.ShapeDtypeStruct((e, HID), jnp.float32),
            jax.ShapeDtypeStruct((e, CWW), jnp.float32),
        ],
    )(hr, hc, pr, pc, wp['w1a'], wp['w1b'], wp['w1d'], wp['b1'],
      wp['w2'], wp['b2'], wp['wc1'], wp['bc1'], wp['wc2t'])


def _node_body(h_ref, aggp_ref, cwp_ref, pos_ref,
               wn1a, wn1b, bn1, wn2, bn2, g, b, mask3, cnt_sel,
               hout_ref, posout_ref):
    h = h_ref[...]
    agg = aggp_ref[0] + aggp_ref[1]
    cuc = cwp_ref[0] + cwp_ref[1]
    nu = _silu(h @ wn1a[...] + agg @ wn1b[...] + bn1[...])
    nu = nu @ wn2[...] + bn2[...]
    x = h + nu
    mu = jnp.mean(x, axis=1, keepdims=True)
    xc = x - mu
    var = jnp.mean(xc * xc, axis=1, keepdims=True)
    hout_ref[...] = xc * lax.rsqrt(var + 1e-5) * g[...] + b[...]
    cnt = jnp.sum(cuc * cnt_sel[...], axis=1, keepdims=True)
    posout_ref[...] = pos_ref[...] + cuc * mask3[...] / (cnt + 1e-6)


def _node_call(h, aggp, cwp, pos16, wp, mask3, cnt_sel, bn):
    n = h.shape[0]
    grid = (n // bn,)
    full = lambda shape: pl.BlockSpec(shape, lambda i: (0, 0))
    return pl.pallas_call(
        _node_body,
        grid=grid,
        in_specs=[
            pl.BlockSpec((bn, HID), lambda i: (i, 0)),
            pl.BlockSpec((NC, bn, HID), lambda i: (0, i, 0)),
            pl.BlockSpec((NC, bn, CWW), lambda i: (0, i, 0)),
            pl.BlockSpec((bn, POSW), lambda i: (i, 0)),
            full((HID, HID)), full((HID, HID)), full((1, HID)),
            full((HID, HID)), full((1, HID)),
            full((1, HID)), full((1, HID)),
            full((1, CWW)), full((1, CWW)),
        ],
        out_specs=[
            pl.BlockSpec((bn, HID), lambda i: (i, 0)),
            pl.BlockSpec((bn, POSW), lambda i: (i, 0)),
        ],
        out_shape=[
            jax.ShapeDtypeStruct((n, HID), jnp.float32),
            jax.ShapeDtypeStruct((n, POSW), jnp.float32),
        ],
    )(h, aggp, cwp, pos16, wp['wn1a'], wp['wn1b'], wp['bn1'],
      wp['wn2'], wp['bn2'], wp['g'], wp['b'], mask3, cnt_sel)


def _pool_body(h_ref, bids_ref, wo0, bo0, wo1, bo1, wo2, bo2,
               out_ref, sums, cnts):
    i = pl.program_id(0)
    nblk = pl.num_programs(0)

    @pl.when(i == 0)
    def _():
        sums[...] = jnp.zeros_like(sums)
        cnts[...] = jnp.zeros_like(cnts)

    bn = h_ref.shape[0]
    bp = sums.shape[0]
    bids = bids_ref[...].reshape(1, bn)
    oh = (lax.broadcasted_iota(jnp.int32, (bp, bn), 0) == bids).astype(jnp.float32)
    sums[...] += oh @ h_ref[...]
    cnts[...] += jnp.sum(oh, axis=1, keepdims=True)

    @pl.when(i == nblk - 1)
    def _():
        gf = sums[...] / jnp.maximum(cnts[...], 1.0)
        gg = _silu(gf @ wo0[...] + bo0[...])
        gg = _silu(gg @ wo1[...] + bo1[...])
        out_ref[...] = gg @ wo2[...] + bo2[...]


def _pool_call(h, bids3, wp, bp, bn):
    n = h.shape[0]
    grid = (n // bn,)
    hh = HID // 2
    full = lambda shape: pl.BlockSpec(shape, lambda i: (0, 0))
    return pl.pallas_call(
        _pool_body,
        grid=grid,
        in_specs=[
            pl.BlockSpec((bn, HID), lambda i: (i, 0)),
            pl.BlockSpec((1, 1, bn), lambda i: (i, 0, 0)),
            full((HID, HID)), full((1, HID)),
            full((HID, hh)), full((1, hh)),
            full((hh, HID)), full((1, HID)),
        ],
        out_specs=pl.BlockSpec((bp, HID), lambda i: (0, 0)),
        out_shape=jax.ShapeDtypeStruct((bp, HID), jnp.float32),
        scratch_shapes=[
            pltpu.VMEM((bp, HID), jnp.float32),
            pltpu.VMEM((bp, 1), jnp.float32),
        ],
    )(h, bids3, wp['wo0'], wp['bo0'], wp['wo1'], wp['bo1'], wp['wo2'], wp['bo2'])


def kernel(pos, atom_types, params, edge_index, batch):
    n = pos.shape[0]
    e = edge_index.shape[1]
    b = 200
    lat = 64
    bn = 1000
    be = 4000
    bp = 256
    nb = e // CH

    row1 = edge_index[0]
    rc = jnp.stack([edge_index[0].reshape(nb, CH),
                    edge_index[1].reshape(nb, CH)], axis=1)
    pos16 = jnp.zeros((n, POSW), jnp.float32).at[:, :3].set(pos)
    zeros_m = jnp.zeros((n, HID), jnp.float32)
    zeros_c = jnp.zeros((n, CWW), jnp.float32)
    bids3 = batch.astype(jnp.int32).reshape(n // bn, 1, bn)

    r2 = lambda v: v.reshape(1, -1)
    emb = params['emb']
    embp = {'w0': emb[0]['W'], 'b0': r2(emb[0]['b']),
            'w1': emb[1]['W'], 'b1': r2(emb[1]['b']),
            'w2': emb[2]['W'], 'b2': r2(emb[2]['b'])}
    h = _emb_call(atom_types, embp, bn)

    iota16 = jnp.arange(CWW)
    mask3 = (iota16 < 3).astype(jnp.float32).reshape(1, CWW)
    cnt_sel = (iota16 == 3).astype(jnp.float32).reshape(1, CWW)

    bf = lambda v: v.astype(jnp.bfloat16)
    for p in params['layers']:
        e1w = p['e1']['W']
        wp_e = {'w1a': bf(e1w[:HID]), 'w1b': bf(e1w[HID:2 * HID]),
                'w1d': e1w[2 * HID:2 * HID + 1], 'b1': r2(p['e1']['b']),
                'w2': bf(p['e2']['W']), 'b2': r2(p['e2']['b']),
                'wc1': bf(p['c1']['W']), 'bc1': r2(p['c1']['b']),
                'wc2t': p['c2']['W'].reshape(1, HID)}
        n1w = p['n1']['W']
        wp_n = {'wn1a': n1w[:HID], 'wn1b': n1w[HID:],
                'bn1': r2(p['n1']['b']), 'wn2': p['n2']['W'],
                'bn2': r2(p['n2']['b']), 'g': r2(p['ln_g']), 'b': r2(p['ln_b'])}

        hr, hc, pr, pc = _sc_gather(h, pos16, rc)
        m, cwx = _edge_call(hr, hc, pr, pc, wp_e, be)
        aggp, cwp = _sc_scatter(m, cwx, row1, zeros_m, zeros_c)
        h, pos16 = _node_call(h, aggp, cwp, pos16, wp_n, mask3, cnt_sel, bn)

    out = params['out']
    wp_o = {'wo0': out[0]['W'], 'bo0': r2(out[0]['b']),
            'wo1': out[1]['W'], 'bo1': r2(out[1]['b']),
            'wo2': out[2]['W'], 'bo2': r2(out[2]['b'])}
    lp = _pool_call(h, bids3, wp_o, bp, bn)
    return lp[:b, :lat], lp[:b, lat:2 * lat]
